# baseline (speedup 1.0000x reference)
import functools

import jax
import jax.numpy as jnp
from jax import lax
from jax.experimental import pallas as pl
from jax.experimental.pallas import tpu as pltpu
from jax.experimental.pallas import tpu_sc as plsc

SEQ = 2048
DIM = 256
BATCH = 4
NS = 16
SHALF = SEQ // 2
S_PER_W = SHALF // NS
NH = 2
HALF = S_PER_W // NH
LANES = 16
DCHUNKS = DIM // LANES


def _emb_body(half_id, x_hbm, tok_hbm, pos_hbm, out_hbm, idx_v, tok_v, pos_v,
              idx_sem, pos_sem, gat_sem, st_sem):
    wid = lax.axis_index("s")
    s0 = half_id * SHALF + wid * S_PER_W
    o0 = wid * S_PER_W

    idx_cps = [
        pltpu.async_copy(x_hbm.at[pl.ds(b * SEQ + s0 + h * HALF, HALF)],
                         idx_v.at[h * BATCH + b], idx_sem)
        for h in range(NH) for b in range(BATCH)
    ]
    pos_cp = pltpu.async_copy(pos_hbm.at[pl.ds(s0, S_PER_W)], pos_v, pos_sem)

    gathers = [None] * (NH * BATCH)
    for k in range(NH * BATCH):
        idx_cps[k].wait()
        gathers[k] = pltpu.async_copy(
            tok_hbm.at[idx_v.at[k]], tok_v.at[k], gat_sem)
    pos_cp.wait()

    stores = []
    for h in range(NH):
        for b in range(BATCH):
            gathers[h * BATCH + b].wait()

        @plsc.parallel_loop(0, HALF)
        def _add(r):
            for c in range(DCHUNKS):
                sl = pl.ds(c * LANES, LANES)
                p = pos_v[h * HALF + r, sl]
                for b in range(BATCH):
                    plsc.addupdate(tok_v.at[h * BATCH + b, r, sl], p)

        for b in range(BATCH):
            stores.append(pltpu.async_copy(
                tok_v.at[h * BATCH + b],
                out_hbm.at[pl.ds(b * SHALF + o0 + h * HALF, HALF)], st_sem))
    for st in stores:
        st.wait()


def kernel(x, token_table, pos_table):
    B, S = x.shape
    xf = x.reshape(B * S).astype(jnp.int32)
    mesh = plsc.VectorSubcoreMesh(core_axis_name="c", subcore_axis_name="s",
                                  num_cores=1)
    scratch = [
        pltpu.VMEM((NH * BATCH, HALF), jnp.int32),
        pltpu.VMEM((NH * BATCH, HALF, DIM), jnp.float32),
        pltpu.VMEM((S_PER_W, DIM), jnp.float32),
        pltpu.SemaphoreType.DMA,
        pltpu.SemaphoreType.DMA,
        pltpu.SemaphoreType.DMA,
        pltpu.SemaphoreType.DMA,
    ]
    outs = []
    for half_id in range(2):
        call = pl.kernel(
            functools.partial(_emb_body, half_id),
            out_type=jax.ShapeDtypeStruct((B * SHALF, DIM), jnp.float32),
            mesh=mesh,
            scratch_types=scratch,
        )
        outs.append(call(xf, token_table, pos_table).reshape(B, SHALF, DIM))
    return jnp.concatenate(outs, axis=1)


# speedup vs baseline: 1.6266x; 1.6266x over previous
import jax
import jax.numpy as jnp
from jax import lax
from jax.experimental import pallas as pl
from jax.experimental.pallas import tpu as pltpu
from jax.experimental.pallas import tpu_sc as plsc

SEQ = 2048
DIM = 256
BATCH = 4
NC = 2
NS = 16
NW = NC * NS
S_PER_W = SEQ // NW
NH = 2
HALF = S_PER_W // NH
LANES = 16
DCHUNKS = DIM // LANES


def _emb_body(x_hbm, tok_hbm, pos_hbm, out_hbm, idx_v, tok_v, pos_v,
              idx_sem, pos_sem, gat_sem, st_sem):
    wid = lax.axis_index("s") * NC + lax.axis_index("c")
    s0 = wid * S_PER_W

    idx_cp = pltpu.async_copy(x_hbm.at[wid], idx_v, idx_sem)
    pos_cp = pltpu.async_copy(pos_hbm.at[pl.ds(s0, S_PER_W)], pos_v, pos_sem)

    gathers = [None] * (NH * BATCH)
    idx_cp.wait()
    for k in range(NH * BATCH):
        gathers[k] = pltpu.async_copy(
            tok_hbm.at[idx_v.at[k]], tok_v.at[k], gat_sem)
    pos_cp.wait()

    stores = []
    for h in range(NH):
        for b in range(BATCH):
            gathers[h * BATCH + b].wait()

        @plsc.parallel_loop(0, HALF)
        def _add(r):
            for c in range(DCHUNKS):
                sl = pl.ds(c * LANES, LANES)
                p = pos_v[h * HALF + r, sl]
                for b in range(BATCH):
                    plsc.addupdate(tok_v.at[h * BATCH + b, r, sl], p)

        for b in range(BATCH):
            stores.append(pltpu.async_copy(
                tok_v.at[h * BATCH + b],
                out_hbm.at[pl.ds(b * SEQ + s0 + h * HALF, HALF)], st_sem))
    for st in stores:
        st.wait()


def kernel(x, token_table, pos_table):
    B, S = x.shape
    xr = (x.astype(jnp.int32)
          .reshape(B, NW, NH, HALF)
          .transpose(1, 2, 0, 3)
          .reshape(NW, NH * BATCH, HALF))
    call = pl.kernel(
        _emb_body,
        out_type=jax.ShapeDtypeStruct((B * S, DIM), jnp.float32),
        mesh=plsc.VectorSubcoreMesh(core_axis_name="c", subcore_axis_name="s"),
        scratch_types=[
            pltpu.VMEM((NH * BATCH, HALF), jnp.int32),
            pltpu.VMEM((NH * BATCH, HALF, DIM), jnp.float32),
            pltpu.VMEM((S_PER_W, DIM), jnp.float32),
            pltpu.SemaphoreType.DMA,
            pltpu.SemaphoreType.DMA,
            pltpu.SemaphoreType.DMA,
            pltpu.SemaphoreType.DMA,
        ],
    )
    out = call(xr, token_table, pos_table)
    return out.reshape(B, S, DIM)


# speedup vs baseline: 1.6422x; 1.0096x over previous
import jax
import jax.numpy as jnp
from jax import lax
from jax.experimental import pallas as pl
from jax.experimental.pallas import tpu as pltpu
from jax.experimental.pallas import tpu_sc as plsc

SEQ = 2048
DIM = 256
BATCH = 4
NC = 2
NS = 16
NW = NC * NS
S_PER_W = SEQ // NW
NQ = BATCH // 2
LANES = 16
DCHUNKS = DIM // LANES


def _emb_body(x_hbm, tok_hbm, pos_hbm, out_hbm, idx_v, tok_v, pos_v,
              idx_sem, pos_sem, gat_sem, st_sem):
    wid = lax.axis_index("s") * NC + lax.axis_index("c")
    s0 = wid * S_PER_W

    idx_cp = pltpu.async_copy(x_hbm.at[wid], idx_v, idx_sem)
    pos_cp = pltpu.async_copy(pos_hbm.at[pl.ds(s0, S_PER_W)], pos_v, pos_sem)

    idx_cp.wait()
    gathers = [
        pltpu.async_copy(tok_hbm.at[idx_v.at[q]], tok_v.at[q], gat_sem)
        for q in range(NQ)
    ]
    pos_cp.wait()

    stores = []
    for q in range(NQ):
        gathers[q].wait()

        @plsc.parallel_loop(0, S_PER_W)
        def _add(r):
            for c in range(DCHUNKS):
                sl = pl.ds(c * LANES, LANES)
                p = pos_v[r, sl]
                plsc.addupdate(tok_v.at[q, r, sl], p)
                plsc.addupdate(tok_v.at[q, S_PER_W + r, sl], p)

        for i in range(2):
            b = 2 * q + i
            stores.append(pltpu.async_copy(
                tok_v.at[q, pl.ds(i * S_PER_W, S_PER_W)],
                out_hbm.at[pl.ds(b * SEQ + s0, S_PER_W)], st_sem))
    for st in stores:
        st.wait()


def kernel(x, token_table, pos_table):
    B, S = x.shape
    xr = (x.astype(jnp.int32)
          .reshape(NQ, 2, NW, S_PER_W)
          .transpose(2, 0, 1, 3)
          .reshape(NW, NQ, 2 * S_PER_W))
    call = pl.kernel(
        _emb_body,
        out_type=jax.ShapeDtypeStruct((B * S, DIM), jnp.float32),
        mesh=plsc.VectorSubcoreMesh(core_axis_name="c", subcore_axis_name="s"),
        scratch_types=[
            pltpu.VMEM((NQ, 2 * S_PER_W), jnp.int32),
            pltpu.VMEM((NQ, 2 * S_PER_W, DIM), jnp.float32),
            pltpu.VMEM((S_PER_W, DIM), jnp.float32),
            pltpu.SemaphoreType.DMA,
            pltpu.SemaphoreType.DMA,
            pltpu.SemaphoreType.DMA,
            pltpu.SemaphoreType.DMA,
        ],
    )
    out = call(xr, token_table, pos_table)
    return out.reshape(B, S, DIM)
